# baseline (speedup 1.0000x reference)
import jax
import jax.numpy as jnp
from jax.experimental import pallas as pl
from jax.experimental.pallas import tpu as pltpu
from jax.experimental.pallas import tpu_sc as plsc

B, N, C, H, E, K = 2, 2048, 1024, 16, 16, 2
Dh = C // H
T = B * N
TB = 512
NTB = T // TB
HP = H // 2
SCALE = Dh ** -0.5
_SQRT_HALF = 0.7071067811865476


def _layernorm(x, g, b):
    m = jnp.mean(x, axis=-1, keepdims=True)
    v = jnp.mean((x - m) ** 2, axis=-1, keepdims=True)
    return (x - m) * jax.lax.rsqrt(v + 1e-5) * g + b


def _ncdf(z):
    return 0.5 * (1.0 + jax.lax.erf(z * _SQRT_HALF))


def _ln_qkv_kernel(x_ref, g_ref, b_ref, w_ref, bias_ref, qkv_ref):
    h = _layernorm(x_ref[...], g_ref[...], b_ref[...])
    qkv_ref[...] = (
        jnp.dot(h.astype(jnp.bfloat16), w_ref[...],
                preferred_element_type=jnp.float32)
        + bias_ref[...]
    ).astype(jnp.bfloat16)


def _attn_kernel(q_ref, k_ref, v_ref, m_ref, o_ref):
    q2 = q_ref[0]
    k2 = k_ref[0]
    v2 = v_ref[0]
    m0 = m_ref[...]
    o2s = []
    for half in (0, 1):
        m = m0 if half == 0 else jnp.bfloat16(1) - m0
        kh = k2 * m
        s = jax.lax.dot_general(
            q2, kh, (((1,), (1,)), ((), ())),
            preferred_element_type=jnp.float32)
        p = jnp.exp2(s).astype(jnp.bfloat16)
        vo = v2 * m + (jnp.bfloat16(1) - m)
        o2s.append(jnp.dot(p, vo, preferred_element_type=jnp.float32))
    mf = m0.astype(jnp.float32)
    nf = 1.0 - mf
    num = o2s[0] * mf + o2s[1] * nf
    den = pltpu.roll(o2s[0] * nf + o2s[1] * mf, 64, 1)
    o_ref[0] = (num / den).astype(jnp.bfloat16)


def _proj_gate_kernel(o_ref, w_ref, b_ref, x_ref, g2_ref, b2_ref,
                      wg_ref, noise_ref,
                      x2_ref, h2_ref, clean_ref, std_ref, noisy_ref):
    x2 = (
        x_ref[...]
        + jnp.dot(o_ref[...], w_ref[...], preferred_element_type=jnp.float32)
        + b_ref[...]
    )
    x2_ref[...] = x2
    h2 = _layernorm(x2, g2_ref[...], b2_ref[...])
    h2b = h2.astype(jnp.bfloat16)
    h2_ref[...] = h2b
    gl = jnp.dot(h2b, wg_ref[...], preferred_element_type=jnp.float32)
    clean, raw = gl[:, :E], gl[:, E:]
    std = jax.nn.softplus(raw) + 1e-2
    clean_ref[...] = clean
    std_ref[...] = std
    noisy_ref[...] = clean + noise_ref[...] * std


def _mlp_kernel(x2_ref, h2_ref, fc1_ref, b1_ref, fc2_ref, b2_ref, y_ref):
    a = jnp.dot(h2_ref[...], fc1_ref[...],
                preferred_element_type=jnp.float32)
    a = a + b1_ref[...]
    a = a * _ncdf(a)
    y_ref[...] = (
        x2_ref[...]
        + jnp.dot(a.astype(jnp.bfloat16), fc2_ref[...],
                  preferred_element_type=jnp.float32)
        + b2_ref[...]
    )


NW = 32
TPW = T // NW


def _ncdf_vec(z):
    x = z * _SQRT_HALF
    ax = jnp.abs(x)
    t = 1.0 / (1.0 + 0.3275911 * ax)
    poly = ((((1.061405429 * t - 1.453152027) * t + 1.421413741) * t
             - 0.284496736) * t + 0.254829592) * t
    y = 1.0 - poly * jnp.exp(-ax * ax)
    erf = jnp.where(x >= 0, y, -y)
    return 0.5 * (1.0 + erf)


def _sc_gate_kernel(noisy_hbm, clean_hbm, std_hbm, imp_hbm, load_hbm,
                    nbuf, cbuf, sbuf, ibuf, lbuf):
    wid = jax.lax.axis_index("s") * 2 + jax.lax.axis_index("c")
    base = wid * TPW * E
    pltpu.sync_copy(noisy_hbm.at[pl.ds(base, TPW * E)], nbuf)
    pltpu.sync_copy(clean_hbm.at[pl.ds(base, TPW * E)], cbuf)
    pltpu.sync_copy(std_hbm.at[pl.ds(base, TPW * E)], sbuf)
    iot = jax.lax.broadcasted_iota(jnp.int32, (E,), 0)
    neg = -1e30

    def bmax(x):
        for sh in (8, 4, 2, 1):
            x = jnp.maximum(x, jnp.take(x, jnp.bitwise_xor(iot, sh)))
        return x

    def bmin(x):
        for sh in (8, 4, 2, 1):
            x = jnp.minimum(x, jnp.take(x, jnp.bitwise_xor(iot, sh)))
        return x

    def body(t, carry):
        imp, load = carry
        row = nbuf[pl.ds(t * E, E)]
        crow = cbuf[pl.ds(t * E, E)]
        srow = sbuf[pl.ds(t * E, E)]
        m1 = bmax(row)
        i1 = bmin(jnp.where(row == m1, iot, E))
        oh1 = iot == i1
        n2 = jnp.where(oh1, neg, row)
        m2 = bmax(n2)
        i2 = bmin(jnp.where(n2 == m2, iot, E))
        oh2 = iot == i2
        m3 = bmax(jnp.where(oh2, neg, n2))
        e2 = jnp.exp(m2 - m1)
        g1 = 1.0 / (1.0 + e2)
        imp = imp + jnp.where(oh1, g1, 0.0) + jnp.where(oh2, 1.0 - g1, 0.0)
        rstd = 1.0 / srow
        p_in = _ncdf_vec((crow - m3) * rstd)
        p_out = _ncdf_vec((crow - m2) * rstd)
        load = load + jnp.where(row > m3, p_in, p_out)
        return imp, load

    imp, load = jax.lax.fori_loop(
        0, TPW, body,
        (jnp.zeros((E,), jnp.float32), jnp.zeros((E,), jnp.float32)))
    ibuf[...] = imp
    lbuf[...] = load
    pltpu.sync_copy(ibuf, imp_hbm.at[pl.ds(wid * E, E)])
    pltpu.sync_copy(lbuf, load_hbm.at[pl.ds(wid * E, E)])


def _loss_kernel(imp_ref, load_ref, out_ref):
    imp = jnp.sum(imp_ref[...], axis=0)
    load = jnp.sum(load_ref[...], axis=0)

    def cv_sq(x):
        m = jnp.mean(x)
        v = jnp.sum((x - m) ** 2) / (E - 1)
        return v / (m * m + 1e-10)

    out_ref[...] = (cv_sq(imp) + cv_sq(load)).reshape(1, 1)


def kernel(x, ln1_g, ln1_b, qkv_w, qkv_b, proj_w, proj_b, ln2_g, ln2_b,
           fc1_w, fc1_b, fc2_w, fc2_b, w_gate, w_noise, noise):
    f32 = jnp.float32
    bf16 = jnp.bfloat16
    xf = x.reshape(T, C)
    row = lambda a: a.reshape(1, -1)
    par = lambda n: pltpu.CompilerParams(
        dimension_semantics=("parallel",) * n)

    qscale = jnp.concatenate(
        [jnp.full((C,), SCALE * 1.4426950408889634, f32),
         jnp.ones((2 * C,), f32)])
    qkv_ws = (qkv_w * qscale).astype(bf16)
    qkv_bs = qkv_b * qscale

    qkv = pl.pallas_call(
        _ln_qkv_kernel,
        grid=(NTB,),
        in_specs=[
            pl.BlockSpec((TB, C), lambda i: (i, 0)),
            pl.BlockSpec((1, C), lambda i: (0, 0)),
            pl.BlockSpec((1, C), lambda i: (0, 0)),
            pl.BlockSpec((C, 3 * C), lambda i: (0, 0)),
            pl.BlockSpec((1, 3 * C), lambda i: (0, 0)),
        ],
        out_specs=pl.BlockSpec((TB, 3 * C), lambda i: (i, 0)),
        out_shape=jax.ShapeDtypeStruct((T, 3 * C), bf16),
        compiler_params=par(1),
    )(xf, row(ln1_g), row(ln1_b), qkv_ws, row(qkv_bs))

    qkv3 = qkv.reshape(B, N, 3 * C)
    halfmask = jnp.concatenate(
        [jnp.ones((1, Dh), bf16), jnp.zeros((1, Dh), bf16)], axis=1)
    o = pl.pallas_call(
        _attn_kernel,
        grid=(B, HP),
        in_specs=[
            pl.BlockSpec((1, N, 2 * Dh), lambda b, j: (b, 0, j)),
            pl.BlockSpec((1, N, 2 * Dh), lambda b, j: (b, 0, HP + j)),
            pl.BlockSpec((1, N, 2 * Dh), lambda b, j: (b, 0, 2 * HP + j)),
            pl.BlockSpec((1, 2 * Dh), lambda b, j: (0, 0)),
        ],
        out_specs=pl.BlockSpec((1, N, 2 * Dh), lambda b, j: (b, 0, j)),
        out_shape=jax.ShapeDtypeStruct((B, N, C), bf16),
        compiler_params=par(2),
    )(qkv3, qkv3, qkv3, halfmask)

    wg = jnp.concatenate([w_gate, w_noise], axis=1).astype(bf16)
    x2, h2, clean, std, noisy = pl.pallas_call(
        _proj_gate_kernel,
        grid=(NTB,),
        in_specs=[
            pl.BlockSpec((TB, C), lambda i: (i, 0)),
            pl.BlockSpec((C, C), lambda i: (0, 0)),
            pl.BlockSpec((1, C), lambda i: (0, 0)),
            pl.BlockSpec((TB, C), lambda i: (i, 0)),
            pl.BlockSpec((1, C), lambda i: (0, 0)),
            pl.BlockSpec((1, C), lambda i: (0, 0)),
            pl.BlockSpec((C, 2 * E), lambda i: (0, 0)),
            pl.BlockSpec((TB, E), lambda i: (i, 0)),
        ],
        out_specs=[
            pl.BlockSpec((TB, C), lambda i: (i, 0)),
            pl.BlockSpec((TB, C), lambda i: (i, 0)),
            pl.BlockSpec((TB, E), lambda i: (i, 0)),
            pl.BlockSpec((TB, E), lambda i: (i, 0)),
            pl.BlockSpec((TB, E), lambda i: (i, 0)),
        ],
        out_shape=[
            jax.ShapeDtypeStruct((T, C), f32),
            jax.ShapeDtypeStruct((T, C), bf16),
            jax.ShapeDtypeStruct((T, E), f32),
            jax.ShapeDtypeStruct((T, E), f32),
            jax.ShapeDtypeStruct((T, E), f32),
        ],
        compiler_params=par(1),
    )(o.reshape(T, C), proj_w.astype(bf16), row(proj_b), xf,
      row(ln2_g), row(ln2_b), wg, noise)

    imp, load = pl.kernel(
        _sc_gate_kernel,
        mesh=plsc.VectorSubcoreMesh(
            core_axis_name="c", subcore_axis_name="s"),
        out_type=[
            jax.ShapeDtypeStruct((NW * E,), f32),
            jax.ShapeDtypeStruct((NW * E,), f32),
        ],
        scratch_types=[
            pltpu.VMEM((TPW * E,), f32),
            pltpu.VMEM((TPW * E,), f32),
            pltpu.VMEM((TPW * E,), f32),
            pltpu.VMEM((E,), f32),
            pltpu.VMEM((E,), f32),
        ],
    )(noisy.reshape(T * E), clean.reshape(T * E), std.reshape(T * E))
    imp = imp.reshape(NW, E)
    load = load.reshape(NW, E)

    y = pl.pallas_call(
        _mlp_kernel,
        grid=(NTB,),
        in_specs=[
            pl.BlockSpec((TB, C), lambda i: (i, 0)),
            pl.BlockSpec((TB, C), lambda i: (i, 0)),
            pl.BlockSpec((C, 4 * C), lambda i: (0, 0)),
            pl.BlockSpec((1, 4 * C), lambda i: (0, 0)),
            pl.BlockSpec((4 * C, C), lambda i: (0, 0)),
            pl.BlockSpec((1, C), lambda i: (0, 0)),
        ],
        out_specs=pl.BlockSpec((TB, C), lambda i: (i, 0)),
        out_shape=jax.ShapeDtypeStruct((T, C), f32),
        compiler_params=par(1),
    )(x2, h2, fc1_w.astype(bf16), row(fc1_b), fc2_w.astype(bf16),
      row(fc2_b))

    loss = pl.pallas_call(
        _loss_kernel,
        in_specs=[
            pl.BlockSpec((NW, E), lambda: (0, 0)),
            pl.BlockSpec((NW, E), lambda: (0, 0)),
        ],
        out_specs=pl.BlockSpec((1, 1), lambda: (0, 0)),
        out_shape=jax.ShapeDtypeStruct((1, 1), f32),
    )(imp, load)

    return y.reshape(B, N, C), loss.reshape(())
